# baseline (speedup 1.0000x reference)
import functools

import jax
import jax.numpy as jnp
from jax import lax
from jax.experimental import pallas as pl
from jax.experimental.pallas import tpu as pltpu
from jax.experimental.pallas import tpu_sc as plsc

_D = 32
_NS = 128
_B = 16384
_L = 16

_info = plsc.get_sparse_core_info()
_NC = _info.num_cores
_NSUB = _info.num_subcores
_NW = _NC * _NSUB
_BPW = _B // _NW
_CHUNK = 128
_NCHUNK = _BPW // _CHUNK


def _sc_body(table_hbm, idx_hbm, func_hbm, params_hbm, idx_v, rows_v, out_v,
             gsem, psem):
    wid = lax.axis_index("s") * _NC + lax.axis_index("c")
    base = wid * _BPW

    pltpu.sync_copy(idx_hbm.at[wid], idx_v)

    copies = [
        pltpu.async_copy(
            table_hbm.at[idx_v.at[j]],
            rows_v.at[pl.ds(j * _CHUNK, _CHUNK)],
            gsem,
        )
        for j in range(_NCHUNK)
    ]
    for c in copies:
        c.wait()

    pcopy = pltpu.async_copy(rows_v, params_hbm.at[pl.ds(base, _BPW)], psem)

    lane = lax.iota(jnp.int32, _L)
    lo_c, hi_c, w_c, om_c = [], [], [], []
    for v in range(_NS // _L):
        num = (lane + v * _L) * (_D - 1)
        lo = lax.div(num, _NS - 1)
        hi = jnp.minimum(lo + 1, _D - 1)
        w = (num - lo * (_NS - 1)).astype(jnp.float32) * (1.0 / (_NS - 1))
        lo_c.append(lo)
        hi_c.append(hi)
        w_c.append(w)
        om_c.append(1.0 - w)

    def row_body(r, carry):
        rvec = jnp.full((_L,), 0, jnp.int32) + r
        for v in range(_NS // _L):
            left = plsc.load_gather(rows_v, [rvec, lo_c[v]])
            right = plsc.load_gather(rows_v, [rvec, hi_c[v]])
            out_v[r, pl.ds(v * _L, _L)] = left * om_c[v] + right * w_c[v]
        return carry

    lax.fori_loop(0, _BPW, row_body, 0)

    pltpu.sync_copy(out_v, func_hbm.at[pl.ds(base, _BPW)])
    pcopy.wait()


@functools.partial(jax.jit, static_argnums=())
def kernel(table, word_indices):
    idx = word_indices.astype(jnp.int32).reshape(_NW, _NCHUNK, _CHUNK)
    mesh = plsc.VectorSubcoreMesh(core_axis_name="c", subcore_axis_name="s")
    call = pl.kernel(
        _sc_body,
        mesh=mesh,
        compiler_params=pltpu.CompilerParams(
            needs_layout_passes=False, use_tc_tiling_on_sc=False),
        out_type=(
            jax.ShapeDtypeStruct((_B, _NS), jnp.float32),
            jax.ShapeDtypeStruct((_B, _D), jnp.float32),
        ),
        scratch_types=[
            pltpu.VMEM((_NCHUNK, _CHUNK), jnp.int32),
            pltpu.VMEM((_BPW, _D), jnp.float32),
            pltpu.VMEM((_BPW, _NS), jnp.float32),
            pltpu.SemaphoreType.DMA,
            pltpu.SemaphoreType.DMA,
        ],
    )
    functions, params = call(table, idx)
    return functions, params


# speedup vs baseline: 2.9009x; 2.9009x over previous
import numpy as np

import jax
import jax.numpy as jnp
from jax import lax
from jax.experimental import pallas as pl
from jax.experimental.pallas import tpu as pltpu
from jax.experimental.pallas import tpu_sc as plsc

_V = 1000000
_VPAD = 1000064
_D = 32
_NS = 128
_B = 16384
_NW = 32
_VPW = 31232
_C = 1024
_NCHUNK = 32
_SMAX = _VPAD - _C
_L = 16
_IB = 2048


def _sc_body(tbl3, idx_hbm, pad_hbm, idxs_v, hits_v, stage_v, rows_v,
             gsem, ssem):
    wid = lax.axis_index("s") * 2 + lax.axis_index("c")
    lo = wid * _VPW
    n_work = jnp.where(wid == _NW - 1, _VPW + 576, _VPW)

    lane = lax.iota(jnp.int32, _L)

    def scan_blk(blk, off):
        pltpu.sync_copy(idx_hbm.at[blk], idxs_v)

        def scan_vec(k, off):
            iv = idxs_v[pl.ds(k * _L, _L)]
            vloc = iv - lo
            m = (vloc >= 0) & (vloc < n_work)
            bpos = blk * _IB + k * _L + lane
            packed = jnp.where(m, (vloc << 14) | bpos, 0)
            cnt = jnp.sum(m.astype(jnp.int32))
            plsc.store_compressed(hits_v.at[pl.ds(off, _L)], packed, mask=m)
            return off + cnt

        return lax.fori_loop(0, _IB // _L, scan_vec, off)

    nh = lax.fori_loop(0, _B // _IB, scan_blk, 0)
    nvec = (nh + _L - 1) // _L

    def do_chunk(ck, _):
        s0 = pl.multiple_of(jnp.minimum(lo + ck * _C, _SMAX), 128)
        cpy = [
            pltpu.async_copy(
                tbl3.at[g, :, pl.ds(s0, _C)], stage_v.at[g], gsem)
            for g in range(4)
        ]
        for cp in cpy:
            cp.wait()

        sl0 = s0 - lo

        def refilt(r, nc):
            h = hits_v[pl.ds(r * _L, _L)]
            active = (r * _L + lane) < nh
            vl = h >> 14
            m = active & (vl >= sl0) & (vl < sl0 + _C)
            cnt = jnp.sum(m.astype(jnp.int32))
            plsc.store_compressed(hits_v.at[pl.ds(_B + nc, _L)], h, mask=m)
            return nc + cnt

        nc = lax.fori_loop(0, nvec, refilt, 0)

        def grp(t, _):
            rem = nc - t * _L
            gm = lane < rem
            h = hits_v[pl.ds(_B + t * _L, _L)]
            h = jnp.where(gm, h, 0)
            v = jnp.where(gm, (h >> 14) - sl0, 0)
            b = jnp.where(gm, h & 0x3FFF, _B + lane)
            zero = jnp.zeros((_L,), jnp.int32)
            for j in range(_D):
                feat = plsc.load_gather(
                    stage_v, [zero + (j // 8), zero + (j % 8), v], mask=gm)
                plsc.store_scatter(rows_v, [lane, zero + j], feat)
            pltpu.async_copy(rows_v.at[:, pl.ds(0, _NS)],
                             pad_hbm.at[b], ssem).wait()
            return 0

        lax.fori_loop(0, (nc + _L - 1) // _L, grp, 0)
        return 0

    lax.fori_loop(0, _NCHUNK, do_chunk, 0)


def _sc_gather(tbl3, idx):
    mesh = plsc.VectorSubcoreMesh(core_axis_name="c", subcore_axis_name="s")
    call = pl.kernel(
        _sc_body,
        mesh=mesh,
        out_type=jax.ShapeDtypeStruct((_B + _L, _NS), jnp.float32),
        scratch_types=[
            pltpu.VMEM((_IB,), jnp.int32),
            pltpu.VMEM((2 * _B,), jnp.int32),
            pltpu.VMEM((4, 8, _C), jnp.float32),
            pltpu.VMEM((_L, _NS), jnp.float32),
            pltpu.SemaphoreType.DMA,
            pltpu.SemaphoreType.DMA,
        ],
        compiler_params=pltpu.CompilerParams(
            needs_layout_passes=False, use_tc_tiling_on_sc=True),
    )
    return call(tbl3, idx)


def _interp_matrices():
    col = np.arange(_NS, dtype=np.int64)
    num = col * (_D - 1)
    lo = num // (_NS - 1)
    hi = np.minimum(lo + 1, _D - 1)
    w = (num - lo * (_NS - 1)).astype(np.float32) / np.float32(_NS - 1)
    m = np.zeros((_NS, _NS), np.float32)
    m[lo, col] += 1.0 - w
    m[hi, col] += w
    sel = np.zeros((_D, _NS), np.float32)
    sel[np.arange(_D), np.arange(_D)] = 1.0
    return jnp.asarray(m), jnp.asarray(sel)


def _tc_body(pad_ref, m_ref, sel_ref, func_ref, pt_ref):
    rows = pad_ref[...]
    func_ref[...] = jax.lax.dot_general(
        rows, m_ref[...], (((1,), (0,)), ((), ())),
        preferred_element_type=jnp.float32)
    pt_ref[...] = jax.lax.dot_general(
        sel_ref[...], rows, (((1,), (1,)), ((), ())),
        preferred_element_type=jnp.float32)


def _tc_interp(pad):
    m, sel = _interp_matrices()
    nblk = _B // 512
    func, pt = pl.pallas_call(
        _tc_body,
        grid=(nblk,),
        in_specs=[
            pl.BlockSpec((512, _NS), lambda i: (i, 0)),
            pl.BlockSpec((_NS, _NS), lambda i: (0, 0)),
            pl.BlockSpec((_D, _NS), lambda i: (0, 0)),
        ],
        out_specs=[
            pl.BlockSpec((512, _NS), lambda i: (i, 0)),
            pl.BlockSpec((_D, 512), lambda i: (0, i)),
        ],
        out_shape=[
            jax.ShapeDtypeStruct((_B, _NS), jnp.float32),
            jax.ShapeDtypeStruct((_D, _B), jnp.float32),
        ],
    )(pad, m, sel)
    return func, pt


def kernel(table, word_indices):
    tbl3 = table.T.reshape(4, 8, _V)
    idx = word_indices.astype(jnp.int32).reshape(_B // _IB, _IB)
    pad = _sc_gather(tbl3, idx)
    func, pt = _tc_interp(pad)
    return func, pt.T
